# baseline (speedup 1.0000x reference)
import functools

import jax
import jax.numpy as jnp
from jax import lax
from jax.experimental import pallas as pl
from jax.experimental.pallas import tpu as pltpu
from jax.experimental.pallas import tpu_sc as plsc

EMBED_DIM = 64
CHUNK = 128


@functools.partial(jax.jit, static_argnames=())
def _embedding_lookup(flat_ids, table):
    B = flat_ids.shape[0]
    info = plsc.get_sparse_core_info()
    num_cores, num_subcores = info.num_cores, info.num_subcores
    nw = num_cores * num_subcores
    b_per_w = B // nw
    n_chunks = b_per_w // CHUNK
    mesh = plsc.VectorSubcoreMesh(core_axis_name="c", subcore_axis_name="s")

    @functools.partial(
        pl.kernel,
        mesh=mesh,
        out_type=jax.ShapeDtypeStruct((B, EMBED_DIM), jnp.float32),
        scratch_types=[
            pltpu.VMEM((CHUNK,), jnp.int32),
            pltpu.VMEM((CHUNK, EMBED_DIM), jnp.float32),
            pltpu.SemaphoreType.DMA,
        ],
        compiler_params=pltpu.CompilerParams(use_tc_tiling_on_sc=False),
    )
    def emb(ids_hbm, table_hbm, out_hbm, idx_v, rows_v, sem):
        wid = lax.axis_index("s") * num_cores + lax.axis_index("c")
        base = wid * b_per_w

        def body(g, carry):
            off = pl.multiple_of(base + g * CHUNK, CHUNK)
            pltpu.sync_copy(ids_hbm.at[pl.ds(off, CHUNK)], idx_v)
            pltpu.async_copy(table_hbm.at[idx_v], rows_v, sem).wait()
            pltpu.sync_copy(rows_v, out_hbm.at[pl.ds(off, CHUNK)])
            return carry

        lax.fori_loop(0, n_chunks, body, 0)

    return emb(flat_ids, table)


def kernel(token_ids, embedding_weights):
    b, t = token_ids.shape
    flat_ids = token_ids.reshape(b * t).astype(jnp.int32)
    out = _embedding_lookup(flat_ids, embedding_weights)
    return out.reshape(b, t, EMBED_DIM)


# speedup vs baseline: 1.1042x; 1.1042x over previous
import functools

import jax
import jax.numpy as jnp
from jax import lax
from jax.experimental import pallas as pl
from jax.experimental.pallas import tpu as pltpu
from jax.experimental.pallas import tpu_sc as plsc

EMBED_DIM = 64
CHUNK = 128
CHUNKS_PER_BLOCK = 4
BLOCK = CHUNK * CHUNKS_PER_BLOCK


@jax.jit
def _embedding_lookup(ids2d, table):
    n_rows_total = ids2d.shape[0] * CHUNK
    info = plsc.get_sparse_core_info()
    num_cores, num_subcores = info.num_cores, info.num_subcores
    nw = num_cores * num_subcores
    chunks_per_w = ids2d.shape[0] // nw
    rows_per_w = chunks_per_w * CHUNK
    n_blocks = chunks_per_w // CHUNKS_PER_BLOCK
    mesh = plsc.VectorSubcoreMesh(core_axis_name="c", subcore_axis_name="s")

    @functools.partial(
        pl.kernel,
        mesh=mesh,
        out_type=jax.ShapeDtypeStruct((n_rows_total, EMBED_DIM), jnp.float32),
        scratch_types=[
            pltpu.VMEM((chunks_per_w, CHUNK), jnp.int32),
            pltpu.VMEM((BLOCK, EMBED_DIM), jnp.float32),
            pltpu.VMEM((BLOCK, EMBED_DIM), jnp.float32),
            pltpu.SemaphoreType.DMA,
            pltpu.SemaphoreType.DMA,
            pltpu.SemaphoreType.DMA,
            pltpu.SemaphoreType.DMA,
        ],
        compiler_params=pltpu.CompilerParams(use_tc_tiling_on_sc=False),
    )
    def emb(ids_hbm, table_hbm, out_hbm, idx_v, buf0, buf1, g0, g1, o0, o1):
        wid = lax.axis_index("s") * num_cores + lax.axis_index("c")
        cbase = wid * chunks_per_w
        rbase = wid * rows_per_w
        bufs = (buf0, buf1)
        gsems = (g0, g1)
        osems = (o0, o1)

        pltpu.sync_copy(ids_hbm.at[pl.ds(cbase, chunks_per_w)], idx_v)

        def fire_gather(g, b):
            for j in range(CHUNKS_PER_BLOCK):
                pltpu.async_copy(
                    table_hbm.at[idx_v.at[g * CHUNKS_PER_BLOCK + j]],
                    bufs[b].at[pl.ds(j * CHUNK, CHUNK)],
                    gsems[b],
                )

        def wait_gather(b):
            pltpu.make_async_copy(
                out_hbm.at[pl.ds(0, BLOCK)], bufs[b], gsems[b]
            ).wait()

        def fire_out(g, b):
            off = pl.multiple_of(rbase + g * BLOCK, BLOCK)
            pltpu.async_copy(bufs[b], out_hbm.at[pl.ds(off, BLOCK)], osems[b])

        def wait_out(b):
            pltpu.make_async_copy(
                bufs[b], out_hbm.at[pl.ds(0, BLOCK)], osems[b]
            ).wait()

        fire_gather(0, 0)
        fire_gather(1, 1)

        def pair(i, carry):
            for b in range(2):
                g = i * 2 + b
                wait_gather(b)
                fire_out(g, b)

                @pl.when(i < n_blocks // 2 - 1)
                def _():
                    wait_out(b)
                    fire_gather(g + 2, b)

            return carry

        lax.fori_loop(0, n_blocks // 2, pair, 0)
        wait_out(0)
        wait_out(1)

    return emb(ids2d, table)


def kernel(token_ids, embedding_weights):
    b, t = token_ids.shape
    ids2d = token_ids.reshape(b * t // CHUNK, CHUNK).astype(jnp.int32)
    out = _embedding_lookup(ids2d, embedding_weights)
    return out.reshape(b, t, EMBED_DIM)
